# Initial kernel scaffold; baseline (speedup 1.0000x reference)
import jax
import jax.numpy as jnp
from jax.experimental import pallas as pl


def kernel(x, edge_index, Wl1, Wr1, b1, Wl2, Wr2, b2):
    raise NotImplementedError("write your pallas kernel here")


# speedup vs baseline: 4.3670x; 4.3670x over previous
import jax
import jax.numpy as jnp
from jax import lax
from jax.experimental import pallas as pl
from jax.experimental.pallas import tpu as pltpu
from jax.experimental.pallas import tpu_sc as plsc

N = 10000
E = 320000
D = 128
H = 128

NC = 2
NS = 16
NW = NC * NS
EPW = E // NW
C = 80
NITER = EPW // C
NP = 10240
RCH = 80
NRCH = NP // RCH
CPT = NRCH // NS

_f32 = jnp.float32


def _zero_acc(sid, zsrc_v, acc_sh):
    def zbody(k, carry):
        j = sid * CPT + k
        pltpu.sync_copy(zsrc_v, acc_sh.at[pl.ds(j * RCH, RCH)])
        return carry

    lax.fori_loop(0, CPT, zbody, 0)


def _copy_out(cid, sid, stage_v, acc_sh, acc_hbm):
    def obody(k, carry):
        j = sid * CPT + k
        pltpu.sync_copy(acc_sh.at[pl.ds(j * RCH, RCH)], stage_v)
        pltpu.sync_copy(stage_v, acc_hbm.at[pl.ds(cid * NP + j * RCH, RCH)])
        return carry

    lax.fori_loop(0, CPT, obody, 0)


def _sc_agg_body(y_hbm, src_hbm, dst_hbm, zrow_hbm, acc_hbm,
                 src_v, dst_v, rows_v, acc_sh, sem):
    cid = lax.axis_index("c")
    sid = lax.axis_index("s")

    pltpu.sync_copy(zrow_hbm, rows_v)
    _zero_acc(sid, rows_v, acc_sh)
    plsc.subcore_barrier()

    ebase = (cid * NS + sid) * EPW

    def ebody(i, carry):
        b = ebase + i * C
        pltpu.sync_copy(src_hbm.at[pl.ds(b, C)], src_v)
        pltpu.sync_copy(dst_hbm.at[pl.ds(b, C)], dst_v)
        pltpu.async_copy(y_hbm.at[src_v], rows_v, sem).wait()
        pltpu.sync_copy(rows_v, acc_sh.at[dst_v], add=True)
        return carry

    lax.fori_loop(0, NITER, ebody, 0)
    plsc.subcore_barrier()

    _copy_out(cid, sid, rows_v, acc_sh, acc_hbm)


def _sc_cnt_body(dst_hbm, zrow_hbm, ones_hbm, cnt_hbm,
                 dst_v, ones_v, stage_v, acc_sh):
    cid = lax.axis_index("c")
    sid = lax.axis_index("s")

    pltpu.sync_copy(zrow_hbm, stage_v)
    pltpu.sync_copy(ones_hbm, ones_v)
    _zero_acc(sid, stage_v, acc_sh)
    plsc.subcore_barrier()

    ebase = (cid * NS + sid) * EPW

    def ebody(i, carry):
        b = ebase + i * C
        pltpu.sync_copy(dst_hbm.at[pl.ds(b, C)], dst_v)
        pltpu.sync_copy(ones_v, acc_sh.at[dst_v], add=True)
        return carry

    lax.fori_loop(0, NITER, ebody, 0)
    plsc.subcore_barrier()

    _copy_out(cid, sid, stage_v, acc_sh, cnt_hbm)


_sc_mesh = plsc.VectorSubcoreMesh(core_axis_name="c", subcore_axis_name="s")

_sc_agg = pl.kernel(
    _sc_agg_body,
    out_type=jax.ShapeDtypeStruct((NC * NP, H), _f32),
    mesh=_sc_mesh,
    scratch_types=[
        pltpu.VMEM((C,), jnp.int32),
        pltpu.VMEM((C,), jnp.int32),
        pltpu.VMEM((C, H), _f32),
        pltpu.VMEM_SHARED((NP, H), _f32),
        pltpu.SemaphoreType.DMA,
    ],
)

_sc_cnt = pl.kernel(
    _sc_cnt_body,
    out_type=jax.ShapeDtypeStruct((NC * NP, H), _f32),
    mesh=_sc_mesh,
    scratch_types=[
        pltpu.VMEM((C,), jnp.int32),
        pltpu.VMEM((C, H), _f32),
        pltpu.VMEM((RCH, H), _f32),
        pltpu.VMEM_SHARED((NP, H), _f32),
    ],
)

_DN = (((1,), (1,)), ((), ()))


def _tc_pre_body(x_ref, wl_ref, wr_ref, b_ref, y_ref, z_ref):
    x = x_ref[...]
    y_ref[...] = lax.dot_general(x, wl_ref[...], _DN,
                                 preferred_element_type=_f32)
    z_ref[...] = lax.dot_general(x, wr_ref[...], _DN,
                                 preferred_element_type=_f32) + b_ref[...]


def _inv_cnt(cnt_ref):
    cnt = cnt_ref[:N, :] + cnt_ref[NP:NP + N, :]
    return 1.0 / jnp.maximum(cnt[:, 0:1], 1.0)


def _tc_mid_body(acc_ref, cnt_ref, z_ref, wl_ref, wr_ref, b_ref,
                 y2_ref, z2_ref):
    acc = acc_ref[:N, :] + acc_ref[NP:NP + N, :]
    h = jnp.maximum(acc * _inv_cnt(cnt_ref) + z_ref[...], 0.0)
    y2_ref[...] = lax.dot_general(h, wl_ref[...], _DN,
                                  preferred_element_type=_f32)
    z2_ref[...] = lax.dot_general(h, wr_ref[...], _DN,
                                  preferred_element_type=_f32) + b_ref[...]


def _tc_post_body(acc_ref, cnt_ref, z_ref, o_ref):
    acc = acc_ref[:N, :] + acc_ref[NP:NP + N, :]
    o_ref[...] = acc * _inv_cnt(cnt_ref) + z_ref[...]


_nh = jax.ShapeDtypeStruct((N, H), _f32)

_tc_pre = pl.pallas_call(_tc_pre_body, out_shape=[_nh, _nh])
_tc_mid = pl.pallas_call(_tc_mid_body, out_shape=[_nh, _nh])
_tc_post = pl.pallas_call(_tc_post_body, out_shape=_nh)


@jax.jit
def kernel(x, edge_index, Wl1, Wr1, b1, Wl2, Wr2, b2):
    src = edge_index[0]
    dst = edge_index[1]
    zrow = jnp.zeros((RCH, H), _f32)
    ones = jnp.ones((C, H), _f32)

    y1, z1 = _tc_pre(x, Wl1, Wr1, b1.reshape(1, H))
    cntp = _sc_cnt(dst, zrow, ones)
    p1 = _sc_agg(y1, src, dst, zrow)
    y2, z2 = _tc_mid(p1, cntp, z1, Wl2, Wr2, b2.reshape(1, H))
    p2 = _sc_agg(y2, src, dst, zrow)
    return _tc_post(p2, cntp, z2)


# speedup vs baseline: 6.3424x; 1.4523x over previous
import jax
import jax.numpy as jnp
from jax import lax
from jax.experimental import pallas as pl
from jax.experimental.pallas import tpu as pltpu
from jax.experimental.pallas import tpu_sc as plsc

N = 10000
E = 320000
D = 128
H = 128

NC = 2
NS = 16
NW = NC * NS
EPW = E // NW
C = 80
NITER = EPW // C
NP = 10240
RCH = 80
NRCH = NP // RCH
CPT = NRCH // NS

_f32 = jnp.float32


def _zero_acc(sid, zsrc_v, acc_sh):
    def zbody(k, carry):
        j = sid * CPT + k
        pltpu.sync_copy(zsrc_v, acc_sh.at[pl.ds(j * RCH, RCH)])
        return carry

    lax.fori_loop(0, CPT, zbody, 0)


def _copy_out(cid, sid, stage_v, acc_sh, acc_hbm):
    def obody(k, carry):
        j = sid * CPT + k
        pltpu.sync_copy(acc_sh.at[pl.ds(j * RCH, RCH)], stage_v)
        pltpu.sync_copy(stage_v, acc_hbm.at[pl.ds(cid * NP + j * RCH, RCH)])
        return carry

    lax.fori_loop(0, CPT, obody, 0)


def _sc_agg_body(y_hbm, src_hbm, dst_hbm, zrow_hbm, acc_hbm,
                 src_a, dst_a, rows_a, src_b, dst_b, rows_b,
                 acc_sh, sem_a, sem_b):
    cid = lax.axis_index("c")
    sid = lax.axis_index("s")

    pltpu.sync_copy(zrow_hbm, rows_a)
    _zero_acc(sid, rows_a, acc_sh)
    plsc.subcore_barrier()

    ebase = (cid * NS + sid) * EPW

    def load_and_gather(i, src_v, dst_v, rows_v, sem):
        b = ebase + i * C
        pltpu.sync_copy(src_hbm.at[pl.ds(b, C)], src_v)
        pltpu.sync_copy(dst_hbm.at[pl.ds(b, C)], dst_v)
        pltpu.async_copy(y_hbm.at[src_v], rows_v, sem)

    def wait_and_scatter(src_v, dst_v, rows_v, sem):
        pltpu.make_async_copy(y_hbm.at[src_v], rows_v, sem).wait()
        pltpu.sync_copy(rows_v, acc_sh.at[dst_v], add=True)

    load_and_gather(0, src_a, dst_a, rows_a, sem_a)

    def ebody(k, carry):
        i = 2 * k
        load_and_gather(i + 1, src_b, dst_b, rows_b, sem_b)
        wait_and_scatter(src_a, dst_a, rows_a, sem_a)
        load_and_gather(i + 2, src_a, dst_a, rows_a, sem_a)
        wait_and_scatter(src_b, dst_b, rows_b, sem_b)
        return carry

    lax.fori_loop(0, (NITER - 1) // 2, ebody, 0)
    wait_and_scatter(src_a, dst_a, rows_a, sem_a)
    plsc.subcore_barrier()

    _copy_out(cid, sid, rows_a, acc_sh, acc_hbm)


def _sc_cnt_body(dst_hbm, zrow_hbm, ones_hbm, cnt_hbm,
                 dst_v, ones_v, stage_v, acc_sh):
    cid = lax.axis_index("c")
    sid = lax.axis_index("s")

    pltpu.sync_copy(zrow_hbm, stage_v)
    pltpu.sync_copy(ones_hbm, ones_v)
    _zero_acc(sid, stage_v, acc_sh)
    plsc.subcore_barrier()

    ebase = (cid * NS + sid) * EPW

    def ebody(i, carry):
        b = ebase + i * C
        pltpu.sync_copy(dst_hbm.at[pl.ds(b, C)], dst_v)
        pltpu.sync_copy(ones_v, acc_sh.at[dst_v], add=True)
        return carry

    lax.fori_loop(0, NITER, ebody, 0)
    plsc.subcore_barrier()

    _copy_out(cid, sid, stage_v, acc_sh, cnt_hbm)


_sc_mesh = plsc.VectorSubcoreMesh(core_axis_name="c", subcore_axis_name="s")

_sc_agg = pl.kernel(
    _sc_agg_body,
    out_type=jax.ShapeDtypeStruct((NC * NP, H), _f32),
    mesh=_sc_mesh,
    scratch_types=[
        pltpu.VMEM((C,), jnp.int32),
        pltpu.VMEM((C,), jnp.int32),
        pltpu.VMEM((C, H), _f32),
        pltpu.VMEM((C,), jnp.int32),
        pltpu.VMEM((C,), jnp.int32),
        pltpu.VMEM((C, H), _f32),
        pltpu.VMEM_SHARED((NP, H), _f32),
        pltpu.SemaphoreType.DMA,
        pltpu.SemaphoreType.DMA,
    ],
)

_sc_cnt = pl.kernel(
    _sc_cnt_body,
    out_type=jax.ShapeDtypeStruct((NC * NP, H), _f32),
    mesh=_sc_mesh,
    scratch_types=[
        pltpu.VMEM((C,), jnp.int32),
        pltpu.VMEM((C, H), _f32),
        pltpu.VMEM((RCH, H), _f32),
        pltpu.VMEM_SHARED((NP, H), _f32),
    ],
)

_DN = (((1,), (1,)), ((), ()))


def _tc_pre_body(x_ref, wl_ref, wr_ref, b_ref, y_ref, z_ref):
    x = x_ref[...]
    y_ref[...] = lax.dot_general(x, wl_ref[...], _DN,
                                 preferred_element_type=_f32)
    z_ref[...] = lax.dot_general(x, wr_ref[...], _DN,
                                 preferred_element_type=_f32) + b_ref[...]


def _inv_cnt(cnt_ref):
    cnt = cnt_ref[:N, :] + cnt_ref[NP:NP + N, :]
    return 1.0 / jnp.maximum(cnt[:, 0:1], 1.0)


def _tc_mid_body(acc_ref, cnt_ref, z_ref, wl_ref, wr_ref, b_ref,
                 y2_ref, z2_ref):
    acc = acc_ref[:N, :] + acc_ref[NP:NP + N, :]
    h = jnp.maximum(acc * _inv_cnt(cnt_ref) + z_ref[...], 0.0)
    y2_ref[...] = lax.dot_general(h, wl_ref[...], _DN,
                                  preferred_element_type=_f32)
    z2_ref[...] = lax.dot_general(h, wr_ref[...], _DN,
                                  preferred_element_type=_f32) + b_ref[...]


def _tc_post_body(acc_ref, cnt_ref, z_ref, o_ref):
    acc = acc_ref[:N, :] + acc_ref[NP:NP + N, :]
    o_ref[...] = acc * _inv_cnt(cnt_ref) + z_ref[...]


_nh = jax.ShapeDtypeStruct((N, H), _f32)

_tc_pre = pl.pallas_call(_tc_pre_body, out_shape=[_nh, _nh])
_tc_mid = pl.pallas_call(_tc_mid_body, out_shape=[_nh, _nh])
_tc_post = pl.pallas_call(_tc_post_body, out_shape=_nh)


@jax.jit
def kernel(x, edge_index, Wl1, Wr1, b1, Wl2, Wr2, b2):
    src = edge_index[0]
    dst = edge_index[1]
    zrow = jnp.zeros((RCH, H), _f32)
    ones = jnp.ones((C, H), _f32)

    y1, z1 = _tc_pre(x, Wl1, Wr1, b1.reshape(1, H))
    cntp = _sc_cnt(dst, zrow, ones)
    p1 = _sc_agg(y1, src, dst, zrow)
    y2, z2 = _tc_mid(p1, cntp, z1, Wl2, Wr2, b2.reshape(1, H))
    p2 = _sc_agg(y2, src, dst, zrow)
    return _tc_post(p2, cntp, z2)


# speedup vs baseline: 7.0448x; 1.1107x over previous
import jax
import jax.numpy as jnp
from jax import lax
from jax.experimental import pallas as pl
from jax.experimental.pallas import tpu as pltpu
from jax.experimental.pallas import tpu_sc as plsc

N = 10000
E = 320000
D = 128
H = 128

NC = 2
NS = 16
NW = NC * NS
EPW = E // NW
C = 80
NITER = EPW // C
NP = 10240
RCH = 80
NRCH = NP // RCH
CPT = NRCH // NS

_f32 = jnp.float32


def _zero_acc(sid, zsrc_v, acc_sh, sem):
    for k in range(CPT):
        j = sid * CPT + k
        pltpu.async_copy(zsrc_v, acc_sh.at[pl.ds(j * RCH, RCH)], sem)
    for k in range(CPT):
        j = sid * CPT + k
        pltpu.make_async_copy(zsrc_v, acc_sh.at[pl.ds(j * RCH, RCH)],
                              sem).wait()


def _copy_out(cid, sid, stage_v, acc_sh, acc_hbm):
    def obody(k, carry):
        j = sid * CPT + k
        pltpu.sync_copy(acc_sh.at[pl.ds(j * RCH, RCH)],
                        acc_hbm.at[pl.ds(cid * NP + j * RCH, RCH)])
        return carry

    lax.fori_loop(0, CPT, obody, 0)


def _sc_agg_body(y_hbm, src_hbm, dst_hbm, zrow_hbm, acc_hbm,
                 src_a, dst_a, rows_a, src_b, dst_b, rows_b,
                 acc_sh, sem_a, sem_b):
    cid = lax.axis_index("c")
    sid = lax.axis_index("s")

    pltpu.sync_copy(zrow_hbm, rows_a)
    _zero_acc(sid, rows_a, acc_sh, sem_a)
    plsc.subcore_barrier()

    ebase = (cid * NS + sid) * EPW

    def load_and_gather(i, src_v, dst_v, rows_v, sem):
        b = ebase + i * C
        pltpu.sync_copy(src_hbm.at[pl.ds(b, C)], src_v)
        pltpu.sync_copy(dst_hbm.at[pl.ds(b, C)], dst_v)
        pltpu.async_copy(y_hbm.at[src_v], rows_v, sem)

    def wait_and_scatter(src_v, dst_v, rows_v, sem):
        pltpu.make_async_copy(y_hbm.at[src_v], rows_v, sem).wait()
        pltpu.sync_copy(rows_v, acc_sh.at[dst_v], add=True)

    load_and_gather(0, src_a, dst_a, rows_a, sem_a)

    def ebody(k, carry):
        i = 2 * k
        load_and_gather(i + 1, src_b, dst_b, rows_b, sem_b)
        wait_and_scatter(src_a, dst_a, rows_a, sem_a)
        load_and_gather(i + 2, src_a, dst_a, rows_a, sem_a)
        wait_and_scatter(src_b, dst_b, rows_b, sem_b)
        return carry

    lax.fori_loop(0, (NITER - 1) // 2, ebody, 0)
    wait_and_scatter(src_a, dst_a, rows_a, sem_a)
    plsc.subcore_barrier()

    _copy_out(cid, sid, rows_a, acc_sh, acc_hbm)


def _sc_cnt_body(dst_hbm, zrow_hbm, ones_hbm, cnt_hbm,
                 dst_a, dst_b, ones_v, stage_v, acc_sh, sem_a, sem_b):
    cid = lax.axis_index("c")
    sid = lax.axis_index("s")

    pltpu.sync_copy(zrow_hbm, stage_v)
    pltpu.sync_copy(ones_hbm, ones_v)
    _zero_acc(sid, stage_v, acc_sh, sem_a)
    plsc.subcore_barrier()

    ebase = (cid * NS + sid) * EPW

    def load(i, dst_v):
        pltpu.sync_copy(dst_hbm.at[pl.ds(ebase + i * C, C)], dst_v)

    def scat(dst_v, sem):
        pltpu.async_copy(ones_v, acc_sh.at[dst_v], sem, add=True)

    def swait(dst_v, sem):
        pltpu.make_async_copy(ones_v, acc_sh.at[dst_v], sem).wait()

    load(0, dst_a)
    scat(dst_a, sem_a)

    def ebody(k, carry):
        i = 2 * k
        load(i + 1, dst_b)
        scat(dst_b, sem_b)
        swait(dst_a, sem_a)
        load(i + 2, dst_a)
        scat(dst_a, sem_a)
        swait(dst_b, sem_b)
        return carry

    lax.fori_loop(0, (NITER - 1) // 2, ebody, 0)
    swait(dst_a, sem_a)
    plsc.subcore_barrier()

    _copy_out(cid, sid, stage_v, acc_sh, cnt_hbm)


_sc_mesh = plsc.VectorSubcoreMesh(core_axis_name="c", subcore_axis_name="s")

_sc_agg = pl.kernel(
    _sc_agg_body,
    out_type=jax.ShapeDtypeStruct((NC * NP, H), _f32),
    mesh=_sc_mesh,
    scratch_types=[
        pltpu.VMEM((C,), jnp.int32),
        pltpu.VMEM((C,), jnp.int32),
        pltpu.VMEM((C, H), _f32),
        pltpu.VMEM((C,), jnp.int32),
        pltpu.VMEM((C,), jnp.int32),
        pltpu.VMEM((C, H), _f32),
        pltpu.VMEM_SHARED((NP, H), _f32),
        pltpu.SemaphoreType.DMA,
        pltpu.SemaphoreType.DMA,
    ],
)

_sc_cnt = pl.kernel(
    _sc_cnt_body,
    out_type=jax.ShapeDtypeStruct((NC * NP, H), _f32),
    mesh=_sc_mesh,
    scratch_types=[
        pltpu.VMEM((C,), jnp.int32),
        pltpu.VMEM((C,), jnp.int32),
        pltpu.VMEM((C, H), _f32),
        pltpu.VMEM((RCH, H), _f32),
        pltpu.VMEM_SHARED((NP, H), _f32),
        pltpu.SemaphoreType.DMA,
        pltpu.SemaphoreType.DMA,
    ],
)

_DN = (((1,), (1,)), ((), ()))


def _tc_pre_body(x_ref, wl_ref, wr_ref, b_ref, y_ref, z_ref):
    x = x_ref[...]
    y_ref[...] = lax.dot_general(x, wl_ref[...], _DN,
                                 preferred_element_type=_f32)
    z_ref[...] = lax.dot_general(x, wr_ref[...], _DN,
                                 preferred_element_type=_f32) + b_ref[...]


def _inv_cnt(cnt_ref):
    cnt = cnt_ref[:N, :] + cnt_ref[NP:NP + N, :]
    return 1.0 / jnp.maximum(cnt[:, 0:1], 1.0)


def _tc_mid_body(acc_ref, cnt_ref, z_ref, wl_ref, wr_ref, b_ref,
                 y2_ref, z2_ref):
    acc = acc_ref[:N, :] + acc_ref[NP:NP + N, :]
    h = jnp.maximum(acc * _inv_cnt(cnt_ref) + z_ref[...], 0.0)
    y2_ref[...] = lax.dot_general(h, wl_ref[...], _DN,
                                  preferred_element_type=_f32)
    z2_ref[...] = lax.dot_general(h, wr_ref[...], _DN,
                                  preferred_element_type=_f32) + b_ref[...]


def _tc_post_body(acc_ref, cnt_ref, z_ref, o_ref):
    acc = acc_ref[:N, :] + acc_ref[NP:NP + N, :]
    o_ref[...] = acc * _inv_cnt(cnt_ref) + z_ref[...]


_nh = jax.ShapeDtypeStruct((N, H), _f32)

_tc_pre = pl.pallas_call(_tc_pre_body, out_shape=[_nh, _nh])
_tc_mid = pl.pallas_call(_tc_mid_body, out_shape=[_nh, _nh])
_tc_post = pl.pallas_call(_tc_post_body, out_shape=_nh)


@jax.jit
def kernel(x, edge_index, Wl1, Wr1, b1, Wl2, Wr2, b2):
    src = edge_index[0]
    dst = edge_index[1]
    zrow = jnp.zeros((RCH, H), _f32)
    ones = jnp.ones((C, H), _f32)

    y1, z1 = _tc_pre(x, Wl1, Wr1, b1.reshape(1, H))
    cntp = _sc_cnt(dst, zrow, ones)
    p1 = _sc_agg(y1, src, dst, zrow)
    y2, z2 = _tc_mid(p1, cntp, z1, Wl2, Wr2, b2.reshape(1, H))
    p2 = _sc_agg(y2, src, dst, zrow)
    return _tc_post(p2, cntp, z2)


# speedup vs baseline: 9.5442x; 1.3548x over previous
import jax
import jax.numpy as jnp
from jax import lax
from jax.experimental import pallas as pl
from jax.experimental.pallas import tpu as pltpu
from jax.experimental.pallas import tpu_sc as plsc

N = 10000
E = 320000
D = 128
H = 128

NC = 2
NS = 16
NW = NC * NS
C = 128
NCH = E // C
FULL = NCH // NW
EXTRA = NCH - FULL * NW
NP = 10240
RCH = 128
NRCH = NP // RCH
CPT = NRCH // NS

_f32 = jnp.float32


def _zero_acc(sid, zsrc_v, acc_sh, sem):
    for k in range(CPT):
        j = sid * CPT + k
        pltpu.async_copy(zsrc_v, acc_sh.at[pl.ds(j * RCH, RCH)], sem)
    for k in range(CPT):
        j = sid * CPT + k
        pltpu.make_async_copy(zsrc_v, acc_sh.at[pl.ds(j * RCH, RCH)],
                              sem).wait()


def _copy_out(cid, sid, acc_sh, acc_hbm):
    def obody(k, carry):
        j = sid * CPT + k
        pltpu.sync_copy(acc_sh.at[pl.ds(j * RCH, RCH)],
                        acc_hbm.at[pl.ds(cid * NP + j * RCH, RCH)])
        return carry

    lax.fori_loop(0, CPT, obody, 0)


def _sc_agg_body(y_hbm, ei_hbm, zrow_hbm, acc_hbm,
                 idx_a, rows_a, idx_b, rows_b,
                 acc_sh, sem_a, sem_b):
    cid = lax.axis_index("c")
    sid = lax.axis_index("s")
    wid = cid * NS + sid

    pltpu.sync_copy(zrow_hbm, rows_a)
    _zero_acc(sid, rows_a, acc_sh, sem_a)
    plsc.subcore_barrier()

    def load_and_gather(i, idx_v, rows_v, sem):
        b = (wid + i * NW) * C
        pltpu.sync_copy(ei_hbm.at[:, pl.ds(b, C)], idx_v)
        pltpu.async_copy(y_hbm.at[idx_v.at[0]], rows_v, sem)

    def wait_and_scatter(idx_v, rows_v, sem):
        pltpu.make_async_copy(y_hbm.at[idx_v.at[0]], rows_v, sem).wait()
        pltpu.sync_copy(rows_v, acc_sh.at[idx_v.at[1]], add=True)

    load_and_gather(0, idx_a, rows_a, sem_a)

    def ebody(k, carry):
        i = 2 * k
        load_and_gather(i + 1, idx_b, rows_b, sem_b)
        wait_and_scatter(idx_a, rows_a, sem_a)
        load_and_gather(i + 2, idx_a, rows_a, sem_a)
        wait_and_scatter(idx_b, rows_b, sem_b)
        return carry

    lax.fori_loop(0, (FULL - 2) // 2, ebody, 0)
    load_and_gather(FULL - 1, idx_b, rows_b, sem_b)
    wait_and_scatter(idx_a, rows_a, sem_a)
    wait_and_scatter(idx_b, rows_b, sem_b)

    @pl.when(wid < EXTRA)
    def _():
        b = (FULL * NW + wid) * C
        pltpu.sync_copy(ei_hbm.at[:, pl.ds(b, C)], idx_a)
        pltpu.async_copy(y_hbm.at[idx_a.at[0]], rows_a, sem_a)
        pltpu.make_async_copy(y_hbm.at[idx_a.at[0]], rows_a, sem_a).wait()
        pltpu.sync_copy(rows_a, acc_sh.at[idx_a.at[1]], add=True)

    plsc.subcore_barrier()
    _copy_out(cid, sid, acc_sh, acc_hbm)


def _sc_cnt_body(ei_hbm, zrow_hbm, ones_hbm, cnt_hbm,
                 idx_a, idx_b, ones_v, stage_v, acc_sh, sem_a, sem_b):
    cid = lax.axis_index("c")
    sid = lax.axis_index("s")
    wid = cid * NS + sid

    pltpu.sync_copy(zrow_hbm, stage_v)
    pltpu.sync_copy(ones_hbm, ones_v)
    _zero_acc(sid, stage_v, acc_sh, sem_a)
    plsc.subcore_barrier()

    def load(i, idx_v):
        b = (wid + i * NW) * C
        pltpu.sync_copy(ei_hbm.at[:, pl.ds(b, C)], idx_v)

    def scat(idx_v, sem):
        pltpu.async_copy(ones_v, acc_sh.at[idx_v.at[1]], sem, add=True)

    def swait(idx_v, sem):
        pltpu.make_async_copy(ones_v, acc_sh.at[idx_v.at[1]], sem).wait()

    load(0, idx_a)
    scat(idx_a, sem_a)

    def ebody(k, carry):
        i = 2 * k
        load(i + 1, idx_b)
        scat(idx_b, sem_b)
        swait(idx_a, sem_a)
        load(i + 2, idx_a)
        scat(idx_a, sem_a)
        swait(idx_b, sem_b)
        return carry

    lax.fori_loop(0, (FULL - 2) // 2, ebody, 0)
    load(FULL - 1, idx_b)
    scat(idx_b, sem_b)
    swait(idx_a, sem_a)
    swait(idx_b, sem_b)

    @pl.when(wid < EXTRA)
    def _():
        b = (FULL * NW + wid) * C
        pltpu.sync_copy(ei_hbm.at[:, pl.ds(b, C)], idx_a)
        pltpu.sync_copy(ones_v, acc_sh.at[idx_a.at[1]], add=True)

    plsc.subcore_barrier()
    _copy_out(cid, sid, acc_sh, cnt_hbm)


_sc_mesh = plsc.VectorSubcoreMesh(core_axis_name="c", subcore_axis_name="s")

_sc_agg = pl.kernel(
    _sc_agg_body,
    out_type=jax.ShapeDtypeStruct((NC * NP, H), _f32),
    mesh=_sc_mesh,
    scratch_types=[
        pltpu.VMEM((2, C), jnp.int32),
        pltpu.VMEM((C, H), _f32),
        pltpu.VMEM((2, C), jnp.int32),
        pltpu.VMEM((C, H), _f32),
        pltpu.VMEM_SHARED((NP, H), _f32),
        pltpu.SemaphoreType.DMA,
        pltpu.SemaphoreType.DMA,
    ],
)

_sc_cnt = pl.kernel(
    _sc_cnt_body,
    out_type=jax.ShapeDtypeStruct((NC * NP, H), _f32),
    mesh=_sc_mesh,
    scratch_types=[
        pltpu.VMEM((2, C), jnp.int32),
        pltpu.VMEM((2, C), jnp.int32),
        pltpu.VMEM((C, H), _f32),
        pltpu.VMEM((RCH, H), _f32),
        pltpu.VMEM_SHARED((NP, H), _f32),
        pltpu.SemaphoreType.DMA,
        pltpu.SemaphoreType.DMA,
    ],
)

_DN = (((1,), (1,)), ((), ()))


def _tc_pre_body(x_ref, wl_ref, wr_ref, b_ref, y_ref, z_ref):
    x = x_ref[...]
    y_ref[...] = lax.dot_general(x, wl_ref[...], _DN,
                                 preferred_element_type=_f32)
    z_ref[...] = lax.dot_general(x, wr_ref[...], _DN,
                                 preferred_element_type=_f32) + b_ref[...]


def _inv_cnt(cnt_ref):
    cnt = cnt_ref[:N, :] + cnt_ref[NP:NP + N, :]
    return 1.0 / jnp.maximum(cnt[:, 0:1], 1.0)


def _tc_mid_body(acc_ref, cnt_ref, z_ref, wl_ref, wr_ref, b_ref,
                 y2_ref, z2_ref):
    acc = acc_ref[:N, :] + acc_ref[NP:NP + N, :]
    h = jnp.maximum(acc * _inv_cnt(cnt_ref) + z_ref[...], 0.0)
    y2_ref[...] = lax.dot_general(h, wl_ref[...], _DN,
                                  preferred_element_type=_f32)
    z2_ref[...] = lax.dot_general(h, wr_ref[...], _DN,
                                  preferred_element_type=_f32) + b_ref[...]


def _tc_post_body(acc_ref, cnt_ref, z_ref, o_ref):
    acc = acc_ref[:N, :] + acc_ref[NP:NP + N, :]
    o_ref[...] = acc * _inv_cnt(cnt_ref) + z_ref[...]


_nh = jax.ShapeDtypeStruct((N, H), _f32)

_tc_pre = pl.pallas_call(_tc_pre_body, out_shape=[_nh, _nh])
_tc_mid = pl.pallas_call(_tc_mid_body, out_shape=[_nh, _nh])
_tc_post = pl.pallas_call(_tc_post_body, out_shape=_nh)


@jax.jit
def kernel(x, edge_index, Wl1, Wr1, b1, Wl2, Wr2, b2):
    zrow = jnp.zeros((RCH, H), _f32)
    ones = jnp.ones((C, H), _f32)

    y1, z1 = _tc_pre(x, Wl1, Wr1, b1.reshape(1, H))
    cntp = _sc_cnt(edge_index, zrow, ones)
    p1 = _sc_agg(y1, edge_index, zrow)
    y2, z2 = _tc_mid(p1, cntp, z1, Wl2, Wr2, b2.reshape(1, H))
    p2 = _sc_agg(y2, edge_index, zrow)
    return _tc_post(p2, cntp, z2)
